# Initial kernel scaffold; baseline (speedup 1.0000x reference)
import jax
import jax.numpy as jnp
from jax.experimental import pallas as pl


def kernel(x, pos_ids, weight):
    raise NotImplementedError("write your pallas kernel here")


# speedup vs baseline: 1.3619x; 1.3619x over previous
import functools

import jax
import jax.numpy as jnp
from jax import lax
from jax.experimental import pallas as pl
from jax.experimental.pallas import tpu as pltpu
from jax.experimental.pallas import tpu_sc as plsc

EMBED = 768
ROWS = 8 * 4608
LANES = 16
NC, NS = 2, 16
NW = NC * NS
RPW = ROWS // NW
CHUNK = 32
NCHUNK = RPW // CHUNK
NBUF = 2

_mesh = plsc.VectorSubcoreMesh(core_axis_name="c", subcore_axis_name="s")


@functools.partial(
    pl.kernel,
    mesh=_mesh,
    out_type=jax.ShapeDtypeStruct((ROWS, EMBED), jnp.float32),
    scratch_types=[
        pltpu.VMEM((NBUF, CHUNK), jnp.int32),
        pltpu.VMEM((NBUF, CHUNK, EMBED), jnp.float32),
        pltpu.VMEM((NBUF, CHUNK, EMBED), jnp.float32),
        pltpu.SemaphoreType.DMA,
        pltpu.SemaphoreType.DMA,
        pltpu.SemaphoreType.DMA,
        pltpu.SemaphoreType.DMA,
    ],
)
def _sc_gather_add(x_hbm, pos_hbm, w_hbm, out_hbm,
                   idx_v, rows_v, x_v, gsem0, gsem1, xsem0, xsem1):
    wid = lax.axis_index("s") * NC + lax.axis_index("c")
    base = wid * RPW
    gsems = (gsem0, gsem1)
    xsems = (xsem0, xsem1)

    def issue(k, b):
        row0 = base + k * CHUNK
        pltpu.sync_copy(pos_hbm.at[pl.ds(row0, CHUNK)], idx_v.at[b])
        pltpu.async_copy(w_hbm.at[idx_v.at[b]], rows_v.at[b], gsems[b])
        pltpu.async_copy(x_hbm.at[pl.ds(row0, CHUNK)], x_v.at[b], xsems[b])

    def waitin(k, b):
        row0 = base + k * CHUNK
        pltpu.make_async_copy(w_hbm.at[idx_v.at[b]], rows_v.at[b], gsems[b]).wait()
        pltpu.make_async_copy(x_hbm.at[pl.ds(row0, CHUNK)], x_v.at[b], xsems[b]).wait()

    for b in range(NBUF):
        issue(b, b)

    def group(g, carry):
        for b in range(NBUF):
            k = g * NBUF + b
            waitin(k, b)

            def row_body(r, c):
                for j in range(EMBED // LANES):
                    sl = pl.ds(j * LANES, LANES)
                    plsc.addupdate(x_v.at[b, r, sl], rows_v[b, r, sl])
                return c

            lax.fori_loop(0, CHUNK, row_body, 0)
            row0 = base + k * CHUNK
            pltpu.sync_copy(x_v.at[b], out_hbm.at[pl.ds(row0, CHUNK)])

            @pl.when(k + NBUF < NCHUNK)
            def _():
                issue(k + NBUF, b)

        return carry

    lax.fori_loop(0, NCHUNK // NBUF, group, 0)


def kernel(x, pos_ids, weight):
    b, l, d = x.shape
    assert b * l == ROWS and d == EMBED
    out = _sc_gather_add(
        x.reshape(ROWS, EMBED), pos_ids.reshape(ROWS), weight
    )
    return out.reshape(x.shape)
